# baseline (speedup 1.0000x reference)
import functools

import jax
import jax.numpy as jnp
from jax import lax
from jax.experimental import pallas as pl
from jax.experimental.pallas import tpu as pltpu
from jax.experimental.pallas import tpu_sc as plsc

B = 8
N = 4096
P = 1024
F = 256
NW = 32
RPW = (B * P) // NW
WPB = P // RPW
GCH = 32
NCH = RPW // GCH
L = 16


def _sc_body(feat_hbm, pts_hbm, idx_hbm, pts_out, feats_out,
             idx_v, gidx_v, rows_v, ptst_v, pouts_v, pnan_v,
             sem_pts, sem_out, *gsems):
    c = lax.axis_index("c")
    s = lax.axis_index("s")
    wid = s * 2 + c
    b = wid // WPB
    chunk = wid % WPB

    pts_cp = []
    for cc in range(3):
        pts_cp.append(pltpu.async_copy(pts_hbm.at[cc, pl.ds(b, 1)],
                                       ptst_v.at[pl.ds(cc, 1)], sem_pts))

    idx_cp = []
    for t in range(2):
        idx_cp.append(pltpu.async_copy(
            idx_hbm.at[pl.ds((chunk * 2 + t) * P + b * 128, 128)],
            idx_v.at[pl.ds(t * 128, 128)], sem_out))
    for cp in idx_cp:
        cp.wait()

    off = b * N
    for i in range(RPW // L):
        v = idx_v[pl.ds(i * L, L)] + off
        gidx_v[i * L // GCH, pl.ds((i * L) % GCH, L)] = v

    copies = []
    for j in range(NCH):
        copies.append(pltpu.async_copy(
            feat_hbm.at[gidx_v.at[j]],
            rows_v.at[pl.ds(j * GCH, GCH)],
            gsems[j]))

    for cp in pts_cp:
        cp.wait()

    def pts_body(i, carry):
        lidx = idx_v[pl.ds(i * L, L)]
        x = plsc.load_gather(ptst_v, [jnp.zeros((L,), jnp.int32), lidx])
        y = plsc.load_gather(ptst_v, [jnp.full((L,), 1, jnp.int32), lidx])
        z = plsc.load_gather(ptst_v, [jnp.full((L,), 2, jnp.int32), lidx])
        pbad = (x != x) | (y != y) | (z != z)
        pouts_v[0, 0, pl.ds(i * L, L)] = x
        pouts_v[1, 0, pl.ds(i * L, L)] = y
        pouts_v[2, 0, pl.ds(i * L, L)] = z
        pnan_v[pl.ds(i * L, L)] = pbad.astype(jnp.int32)
        return carry

    lax.fori_loop(0, RPW // L, pts_body, 0)

    def row_group(i, carry):
        pnanvec = pnan_v[pl.ds(i * L, L)]
        grp = pnanvec != 0
        for r2 in range(L):
            r = i * L + r2
            m = None
            for k in range(F // L):
                f = rows_v[r, pl.ds(k * L, L)]
                m = f if m is None else jnp.maximum(m, f)
            grp = grp | (m != m)
        cnt = plsc.all_reduce_population_count(grp)

        @pl.when(cnt[0] != 0)
        def _slow():
            def fix_row(r, carry2):
                m = None
                for k in range(F // L):
                    f = rows_v[r, pl.ds(k * L, L)]
                    m = f if m is None else jnp.maximum(m, f)
                ncnt = plsc.all_reduce_population_count(m != m)
                pn = plsc.load_gather(pnan_v, [jnp.full((L,), r, jnp.int32)])
                bad = (ncnt[0] != 0) | (pn[0] != 0)

                @pl.when(bad)
                def _zero():
                    zf = jnp.zeros((L,), jnp.float32)
                    for k in range(F // L):
                        rows_v[r, pl.ds(k * L, L)] = zf
                    lanes = lax.iota(jnp.int32, L)
                    plsc.store_scatter(
                        pouts_v, [lanes, jnp.zeros((L,), jnp.int32),
                                  jnp.full((L,), r, jnp.int32)], zf,
                        mask=lanes < 3)

                return carry2

            lax.fori_loop(i * L, (i + 1) * L, fix_row, 0)

        return carry

    out_copies = []
    for j in range(NCH):
        copies[j].wait()
        lax.fori_loop(j * (GCH // L), (j + 1) * (GCH // L), row_group, 0)
        out_copies.append(pltpu.async_copy(
            rows_v.at[pl.ds(j * GCH, GCH)],
            feats_out.at[b, pl.ds(chunk * RPW + j * GCH, GCH)],
            sem_out))

    for cc in range(3):
        pltpu.sync_copy(
            pouts_v.at[pl.ds(cc, 1)],
            pts_out.at[pl.ds(cc, 1), pl.ds(b, 1), pl.ds(chunk * RPW, RPW)])
    for cp in out_copies:
        cp.wait()


@jax.jit
def kernel(points, features, sample_idx):
    feat2d = features.reshape(B * N, F)
    ptsp = jnp.transpose(points, (2, 0, 1))
    idxp = (sample_idx.astype(jnp.int32)
            .reshape(B, B, 128).transpose(1, 0, 2).reshape(B * P))

    run = functools.partial(
        pl.kernel,
        out_type=(
            jax.ShapeDtypeStruct((3, B, P), jnp.float32),
            jax.ShapeDtypeStruct((B, P, F), jnp.float32),
        ),
        mesh=plsc.VectorSubcoreMesh(core_axis_name="c", subcore_axis_name="s"),
        scratch_types=[
            pltpu.VMEM((RPW,), jnp.int32),
            pltpu.VMEM((NCH, GCH), jnp.int32),
            pltpu.VMEM((RPW, F), jnp.float32),
            pltpu.VMEM((3, N), jnp.float32),
            pltpu.VMEM((3, 1, RPW), jnp.float32),
            pltpu.VMEM((RPW,), jnp.int32),
            pltpu.SemaphoreType.DMA,
            pltpu.SemaphoreType.DMA,
        ] + [pltpu.SemaphoreType.DMA] * NCH,
        compiler_params=pltpu.CompilerParams(needs_layout_passes=False),
    )(_sc_body)

    pts_planes, feats_ds = run(feat2d, ptsp, idxp)
    pts_ds = jnp.transpose(pts_planes, (1, 2, 0))
    return pts_ds, feats_ds


# speedup vs baseline: 1.0953x; 1.0953x over previous
import functools

import jax
import jax.numpy as jnp
from jax import lax
from jax.experimental import pallas as pl
from jax.experimental.pallas import tpu as pltpu
from jax.experimental.pallas import tpu_sc as plsc

B = 8
N = 4096
P = 1024
F = 256
NW = 32
RPW = (B * P) // NW
WPB = P // RPW
GCH = 128
NCH = RPW // GCH
L = 16


def _sc_body(feat_hbm, pts_hbm, idx_hbm, pts_out, feats_out,
             idx_v, gidx_v, rows_v, ptst_v, pouts_v, pnan_v,
             sem_pts, sem_out, *gsems):
    c = lax.axis_index("c")
    s = lax.axis_index("s")
    wid = s * 2 + c
    b = wid // WPB
    chunk = wid % WPB

    pts_cp = []
    for cc in range(3):
        pts_cp.append(pltpu.async_copy(pts_hbm.at[cc, pl.ds(b, 1)],
                                       ptst_v.at[pl.ds(cc, 1)], sem_pts))

    idx_cp = []
    for t in range(2):
        idx_cp.append(pltpu.async_copy(
            idx_hbm.at[pl.ds((chunk * 2 + t) * P + b * 128, 128)],
            idx_v.at[pl.ds(t * 128, 128)], sem_out))
    for cp in idx_cp:
        cp.wait()

    off = b * N
    for i in range(RPW // L):
        v = idx_v[pl.ds(i * L, L)] + off
        gidx_v[i * L // GCH, pl.ds((i * L) % GCH, L)] = v

    copies = []
    for j in range(NCH):
        copies.append(pltpu.async_copy(
            feat_hbm.at[gidx_v.at[j]],
            rows_v.at[pl.ds(j * GCH, GCH)],
            gsems[j]))

    for cp in pts_cp:
        cp.wait()

    def pts_body(i, carry):
        lidx = idx_v[pl.ds(i * L, L)]
        x = plsc.load_gather(ptst_v, [jnp.zeros((L,), jnp.int32), lidx])
        y = plsc.load_gather(ptst_v, [jnp.full((L,), 1, jnp.int32), lidx])
        z = plsc.load_gather(ptst_v, [jnp.full((L,), 2, jnp.int32), lidx])
        pbad = (x != x) | (y != y) | (z != z)
        pouts_v[0, 0, pl.ds(i * L, L)] = x
        pouts_v[1, 0, pl.ds(i * L, L)] = y
        pouts_v[2, 0, pl.ds(i * L, L)] = z
        pnan_v[pl.ds(i * L, L)] = pbad.astype(jnp.int32)
        return carry

    lax.fori_loop(0, RPW // L, pts_body, 0)

    def row_group(i, carry):
        pnanvec = pnan_v[pl.ds(i * L, L)]
        grp = pnanvec != 0
        for r2 in range(L):
            r = i * L + r2
            m = None
            for k in range(F // L):
                f = rows_v[r, pl.ds(k * L, L)]
                m = f if m is None else jnp.maximum(m, f)
            grp = grp | (m != m)
        cnt = plsc.all_reduce_population_count(grp)

        @pl.when(cnt[0] != 0)
        def _slow():
            def fix_row(r, carry2):
                m = None
                for k in range(F // L):
                    f = rows_v[r, pl.ds(k * L, L)]
                    m = f if m is None else jnp.maximum(m, f)
                ncnt = plsc.all_reduce_population_count(m != m)
                pn = plsc.load_gather(pnan_v, [jnp.full((L,), r, jnp.int32)])
                bad = (ncnt[0] != 0) | (pn[0] != 0)

                @pl.when(bad)
                def _zero():
                    zf = jnp.zeros((L,), jnp.float32)
                    for k in range(F // L):
                        rows_v[r, pl.ds(k * L, L)] = zf
                    lanes = lax.iota(jnp.int32, L)
                    plsc.store_scatter(
                        pouts_v, [lanes, jnp.zeros((L,), jnp.int32),
                                  jnp.full((L,), r, jnp.int32)], zf,
                        mask=lanes < 3)

                return carry2

            lax.fori_loop(i * L, (i + 1) * L, fix_row, 0)

        return carry

    out_copies = []
    for j in range(NCH):
        copies[j].wait()
        lax.fori_loop(j * (GCH // L), (j + 1) * (GCH // L), row_group, 0)
        out_copies.append(pltpu.async_copy(
            rows_v.at[pl.ds(j * GCH, GCH)],
            feats_out.at[b, pl.ds(chunk * RPW + j * GCH, GCH)],
            sem_out))

    for cc in range(3):
        pltpu.sync_copy(
            pouts_v.at[pl.ds(cc, 1)],
            pts_out.at[pl.ds(cc, 1), pl.ds(b, 1), pl.ds(chunk * RPW, RPW)])
    for cp in out_copies:
        cp.wait()


@jax.jit
def kernel(points, features, sample_idx):
    feat2d = features.reshape(B * N, F)
    ptsp = jnp.transpose(points, (2, 0, 1))
    idxp = (sample_idx.astype(jnp.int32)
            .reshape(B, B, 128).transpose(1, 0, 2).reshape(B * P))

    run = functools.partial(
        pl.kernel,
        out_type=(
            jax.ShapeDtypeStruct((3, B, P), jnp.float32),
            jax.ShapeDtypeStruct((B, P, F), jnp.float32),
        ),
        mesh=plsc.VectorSubcoreMesh(core_axis_name="c", subcore_axis_name="s"),
        scratch_types=[
            pltpu.VMEM((RPW,), jnp.int32),
            pltpu.VMEM((NCH, GCH), jnp.int32),
            pltpu.VMEM((RPW, F), jnp.float32),
            pltpu.VMEM((3, N), jnp.float32),
            pltpu.VMEM((3, 1, RPW), jnp.float32),
            pltpu.VMEM((RPW,), jnp.int32),
            pltpu.SemaphoreType.DMA,
            pltpu.SemaphoreType.DMA,
        ] + [pltpu.SemaphoreType.DMA] * NCH,
        compiler_params=pltpu.CompilerParams(needs_layout_passes=False,
                                             skip_device_barrier=True),
    )(_sc_body)

    pts_planes, feats_ds = run(feat2d, ptsp, idxp)
    pts_ds = jnp.transpose(pts_planes, (1, 2, 0))
    return pts_ds, feats_ds
